# baseline (speedup 1.0000x reference)
import jax
import jax.numpy as jnp
from jax.experimental import pallas as pl
from jax.experimental.pallas import tpu as pltpu

N = 10000
K = 32
D = 128
P = 8
B = 256

A = 400
NCHUNK = N // A
KD = K * D
W = 32
J = 2 * K
BP = B + W


def _silu(x):
    return x * jax.nn.sigmoid(x)


def _fused_kernel(meta_ref, x_ref, kdrt_ref, sinct_ref, batch_ref, bcol_ref,
                  w1t_ref, w2t_ref, gamma_ref, beta_ref, kf3_ref,
                  e1t_ref, e1b_ref, wu1t_ref, wu2t_ref,
                  out_ref, sf_acc, g_s, cbsb_s):
    i = pl.program_id(0)

    @pl.when(i == 0)
    def _init():
        sf_acc[...] = jnp.zeros_like(sf_acc)

    @pl.when(i < NCHUNK)
    def _phase_a():
        ia = jnp.minimum(i, NCHUNK - 1)
        x = x_ref[...]
        h = _silu(jnp.dot(x, w1t_ref[...], preferred_element_type=jnp.float32))
        h = _silu(jnp.dot(h, w2t_ref[...], preferred_element_type=jnp.float32))
        xr = x + h
        mean = jnp.mean(xr, axis=-1, keepdims=True)
        var = jnp.mean((xr - mean) ** 2, axis=-1, keepdims=True)
        xr = (xr - mean) * jax.lax.rsqrt(var + 1e-5) * gamma_ref[...] \
            + beta_ref[...]
        xrb = xr.astype(jnp.bfloat16)

        sinct = sinct_ref[...].T
        kdrt = kdrt_ref[...].T
        ct = (jnp.cos(kdrt) * sinct).astype(jnp.bfloat16)
        st = (jnp.sin(kdrt) * sinct).astype(jnp.bfloat16)
        ctst = jnp.concatenate([ct, st], axis=0)
        row0 = pl.multiple_of(ia * A, 16)
        cbsb_s[pl.ds(row0, A), :] = ctst.T

        coef_exp = jnp.dot(e1t_ref[...], ctst,
                           preferred_element_type=jnp.float32
                           ).astype(jnp.bfloat16)

        bvec = batch_ref[0]
        base = meta_ref[0, ia]
        last = meta_ref[1, ia]

        def _scatter_pass(off):
            ot = (jax.lax.broadcasted_iota(jnp.int32, (W, A), 0) + off
                  == bvec).astype(jnp.bfloat16)
            mt = coef_exp * jnp.concatenate([ot] * J, axis=0)
            res = jnp.dot(mt, xrb,
                          preferred_element_type=jnp.float32)
            off8 = pl.multiple_of(off, 16)
            sf_acc[:, pl.ds(off8, W), :] += res.reshape(J, W, D)

        _scatter_pass(base)

        def _body(p, carry):
            _scatter_pass(base + p * W)
            return carry

        npass = (last - base) // W + 1
        jax.lax.fori_loop(1, npass, _body, jnp.int32(0))

        @pl.when(i == NCHUNK - 1)
        def _emit():
            g_s[...] = (kf3_ref[...] * sf_acc[...]).astype(jnp.bfloat16)

    @pl.when(i >= NCHUNK)
    def _phase_b():
        ib = jnp.maximum(i - NCHUNK, 0)
        row0 = pl.multiple_of(ib * A, 16)
        cbsb = cbsb_s[pl.ds(row0, A), :]
        bcol = bcol_ref[0]
        base = meta_ref[0, ib]
        last = meta_ref[1, ib]

        coef_exp = jnp.dot(cbsb, e1b_ref[...],
                           preferred_element_type=jnp.float32
                           ).astype(jnp.bfloat16)
        lane = jax.lax.broadcasted_iota(jnp.int32, (A, 128), 1)

        def _gather_pass(off):
            ow = ((lane & (W - 1)) + off == bcol
                  ).astype(jnp.bfloat16)
            m = coef_exp * jnp.concatenate(
                [ow] * (J * W // 128), axis=1)
            off8 = pl.multiple_of(off, 16)
            gwin = g_s[:, pl.ds(off8, W), :].reshape(J * W, D)
            return jnp.dot(m, gwin,
                           preferred_element_type=jnp.float32)

        ew = _gather_pass(base)

        def _body(p, acc):
            return acc + _gather_pass(base + p * W)

        npass = (last - base) // W + 1
        ew = jax.lax.fori_loop(1, npass, _body, ew)

        x_new = x_ref[...] + ew
        u = _silu(jnp.dot(x_new, wu1t_ref[...],
                          preferred_element_type=jnp.float32))
        u = _silu(jnp.dot(u, wu2t_ref[...],
                          preferred_element_type=jnp.float32))
        out_ref[...] = x_new + u


@jax.jit
def kernel(x_scalar, k_dot_r, sinc_damping, batch, down_projection,
           W_pre1, W_pre2, gamma, beta, W_up, W_upd1, W_upd2):
    batch_row = batch.reshape(NCHUNK, 1, A)
    batch_col = batch.reshape(NCHUNK, A, 1)
    base = (batch[::A] // 16) * 16
    last = batch[A - 1::A]
    meta = jnp.stack([base, last]).astype(jnp.int32)
    kf = down_projection @ W_up.T
    kf3 = jnp.repeat(kf, 2, axis=0).reshape(J, 1, D)
    gamma2 = gamma.reshape(1, D)
    beta2 = beta.reshape(1, D)

    q_of_row = jnp.arange(J * W, dtype=jnp.int32) // W
    src_row = (q_of_row // 2) + (q_of_row % 2) * K
    e1t = (src_row[:, None] == jnp.arange(J, dtype=jnp.int32)[None, :]
           ).astype(jnp.bfloat16)
    col = jnp.arange(J * W, dtype=jnp.int32)
    q_of_col = (col // 128) * (128 // W) + (col % 128) // W
    src_col = (q_of_col // 2) + (q_of_col % 2) * K
    e1b = (jnp.arange(J, dtype=jnp.int32)[:, None] == src_col[None, :]
           ).astype(jnp.bfloat16)

    chunk = lambda i, m: (i % NCHUNK, 0)
    chunk3 = lambda i, m: (i % NCHUNK, 0, 0)
    whole = lambda i, m: (0, 0)

    out = pl.pallas_call(
        _fused_kernel,
        grid_spec=pltpu.PrefetchScalarGridSpec(
            num_scalar_prefetch=1,
            grid=(2 * NCHUNK,),
            in_specs=[
                pl.BlockSpec((A, D), chunk),
                pl.BlockSpec((A, K), chunk),
                pl.BlockSpec((A, K), chunk),
                pl.BlockSpec((1, 1, A), chunk3),
                pl.BlockSpec((1, A, 1), chunk3),
                pl.BlockSpec((D, D), whole),
                pl.BlockSpec((D, D), whole),
                pl.BlockSpec((1, D), whole),
                pl.BlockSpec((1, D), whole),
                pl.BlockSpec((J, 1, D), lambda i, m: (0, 0, 0)),
                pl.BlockSpec((J * W, J), whole),
                pl.BlockSpec((J, J * W), whole),
                pl.BlockSpec((D, D), whole),
                pl.BlockSpec((D, D), whole),
            ],
            out_specs=pl.BlockSpec(
                (A, D), lambda i, m: (jnp.maximum(i - NCHUNK, 0), 0)),
            scratch_shapes=[
                pltpu.VMEM((J, BP, D), jnp.float32),
                pltpu.VMEM((J, BP, D), jnp.bfloat16),
                pltpu.VMEM((N, J), jnp.bfloat16),
            ],
        ),
        out_shape=jax.ShapeDtypeStruct((N, D), jnp.float32),
    )(meta, x_scalar, k_dot_r, sinc_damping,
      batch_row, batch_col, W_pre1.T, W_pre2.T, gamma2, beta2, kf3,
      e1t, e1b, W_upd1.T, W_upd2.T)

    return out


# speedup vs baseline: 1.0311x; 1.0311x over previous
import jax
import jax.numpy as jnp
from jax.experimental import pallas as pl
from jax.experimental.pallas import tpu as pltpu

N = 10000
K = 32
D = 128
P = 8
B = 256

A = 400
NCHUNK = N // A
KD = K * D
W = 32
J = 2 * K
BP = B + W


def _silu(x):
    return x * jax.nn.sigmoid(x)


def _fused_kernel(meta_ref, x_ref, kdrt_ref, sinct_ref, batch_ref, bcol_ref,
                  w1t_ref, w2t_ref, gamma_ref, beta_ref, kf3_ref,
                  e1t_ref, e1b_ref, wu1t_ref, wu2t_ref,
                  out_ref, sf_acc, g_s, cbsb_s):
    i = pl.program_id(0)

    @pl.when(i == 0)
    def _init():
        sf_acc[...] = jnp.zeros_like(sf_acc)

    @pl.when(i < NCHUNK)
    def _phase_a():
        ia = jnp.minimum(i, NCHUNK - 1)
        x = x_ref[...]
        h = _silu(jnp.dot(x, w1t_ref[...], preferred_element_type=jnp.float32))
        h = _silu(jnp.dot(h, w2t_ref[...], preferred_element_type=jnp.float32))
        xr = x + h
        mean = jnp.mean(xr, axis=-1, keepdims=True)
        var = jnp.mean((xr - mean) ** 2, axis=-1, keepdims=True)
        xr = (xr - mean) * jax.lax.rsqrt(var + 1e-5) * gamma_ref[...] \
            + beta_ref[...]
        xrb = xr.astype(jnp.bfloat16)

        sinct = sinct_ref[0]
        kdrt = kdrt_ref[0]
        ct = (jnp.cos(kdrt) * sinct).astype(jnp.bfloat16)
        st = (jnp.sin(kdrt) * sinct).astype(jnp.bfloat16)
        ctst = jnp.concatenate([ct, st], axis=0)
        row0 = pl.multiple_of(ia * A, 16)
        cbsb_s[pl.ds(row0, A), :] = ctst.T

        coef_exp = jnp.dot(e1t_ref[...], ctst,
                           preferred_element_type=jnp.float32
                           ).astype(jnp.bfloat16)

        bvec = batch_ref[0]
        base = meta_ref[0, ia]
        last = meta_ref[1, ia]

        def _scatter_pass(off):
            ot = (jax.lax.broadcasted_iota(jnp.int32, (W, A), 0) + off
                  == bvec).astype(jnp.bfloat16)
            mt = coef_exp * jnp.concatenate([ot] * J, axis=0)
            res = jnp.dot(mt, xrb,
                          preferred_element_type=jnp.float32)
            off8 = pl.multiple_of(off, 16)
            sf_acc[:, pl.ds(off8, W), :] += res.reshape(J, W, D)

        _scatter_pass(base)

        def _body(p, carry):
            _scatter_pass(base + p * W)
            return carry

        npass = (last - base) // W + 1
        jax.lax.fori_loop(1, npass, _body, jnp.int32(0))

        @pl.when(i == NCHUNK - 1)
        def _emit():
            g_s[...] = (kf3_ref[...] * sf_acc[...]).astype(jnp.bfloat16)

    @pl.when(i >= NCHUNK)
    def _phase_b():
        ib = jnp.maximum(i - NCHUNK, 0)
        row0 = pl.multiple_of(ib * A, 16)
        cbsb = cbsb_s[pl.ds(row0, A), :]
        bcol = bcol_ref[0]
        base = meta_ref[0, ib]
        last = meta_ref[1, ib]

        coef_exp = jnp.dot(cbsb, e1b_ref[...],
                           preferred_element_type=jnp.float32
                           ).astype(jnp.bfloat16)
        lane = jax.lax.broadcasted_iota(jnp.int32, (A, 128), 1)

        def _gather_pass(off):
            ow = ((lane & (W - 1)) + off == bcol
                  ).astype(jnp.bfloat16)
            m = coef_exp * jnp.concatenate(
                [ow] * (J * W // 128), axis=1)
            off8 = pl.multiple_of(off, 16)
            gwin = g_s[:, pl.ds(off8, W), :].reshape(J * W, D)
            return jnp.dot(m, gwin,
                           preferred_element_type=jnp.float32)

        ew = _gather_pass(base)

        def _body(p, acc):
            return acc + _gather_pass(base + p * W)

        npass = (last - base) // W + 1
        ew = jax.lax.fori_loop(1, npass, _body, ew)

        x_new = x_ref[...] + ew
        u = _silu(jnp.dot(x_new, wu1t_ref[...],
                          preferred_element_type=jnp.float32))
        u = _silu(jnp.dot(u, wu2t_ref[...],
                          preferred_element_type=jnp.float32))
        out_ref[...] = x_new + u


@jax.jit
def kernel(x_scalar, k_dot_r, sinc_damping, batch, down_projection,
           W_pre1, W_pre2, gamma, beta, W_up, W_upd1, W_upd2):
    batch_row = batch.reshape(NCHUNK, 1, A)
    batch_col = batch.reshape(NCHUNK, A, 1)
    base = (batch[::A] // 16) * 16
    last = batch[A - 1::A]
    meta = jnp.stack([base, last]).astype(jnp.int32)
    kf = down_projection @ W_up.T
    kf3 = jnp.repeat(kf, 2, axis=0).reshape(J, 1, D)
    gamma2 = gamma.reshape(1, D)
    beta2 = beta.reshape(1, D)

    q_of_row = jnp.arange(J * W, dtype=jnp.int32) // W
    src_row = (q_of_row // 2) + (q_of_row % 2) * K
    e1t = (src_row[:, None] == jnp.arange(J, dtype=jnp.int32)[None, :]
           ).astype(jnp.bfloat16)
    col = jnp.arange(J * W, dtype=jnp.int32)
    q_of_col = (col // 128) * (128 // W) + (col % 128) // W
    src_col = (q_of_col // 2) + (q_of_col % 2) * K
    e1b = (jnp.arange(J, dtype=jnp.int32)[:, None] == src_col[None, :]
           ).astype(jnp.bfloat16)

    chunk = lambda i, m: (i % NCHUNK, 0)
    chunk3 = lambda i, m: (i % NCHUNK, 0, 0)
    whole = lambda i, m: (0, 0)

    out = pl.pallas_call(
        _fused_kernel,
        grid_spec=pltpu.PrefetchScalarGridSpec(
            num_scalar_prefetch=1,
            grid=(2 * NCHUNK,),
            in_specs=[
                pl.BlockSpec((A, D), chunk),
                pl.BlockSpec((1, K, A), chunk3),
                pl.BlockSpec((1, K, A), chunk3),
                pl.BlockSpec((1, 1, A), chunk3),
                pl.BlockSpec((1, A, 1), chunk3),
                pl.BlockSpec((D, D), whole),
                pl.BlockSpec((D, D), whole),
                pl.BlockSpec((1, D), whole),
                pl.BlockSpec((1, D), whole),
                pl.BlockSpec((J, 1, D), lambda i, m: (0, 0, 0)),
                pl.BlockSpec((J * W, J), whole),
                pl.BlockSpec((J, J * W), whole),
                pl.BlockSpec((D, D), whole),
                pl.BlockSpec((D, D), whole),
            ],
            out_specs=pl.BlockSpec(
                (A, D), lambda i, m: (jnp.maximum(i - NCHUNK, 0), 0)),
            scratch_shapes=[
                pltpu.VMEM((J, BP, D), jnp.float32),
                pltpu.VMEM((J, BP, D), jnp.bfloat16),
                pltpu.VMEM((N, J), jnp.bfloat16),
            ],
        ),
        out_shape=jax.ShapeDtypeStruct((N, D), jnp.float32),
    )(meta, x_scalar,
      k_dot_r.T.reshape(K, NCHUNK, A).transpose(1, 0, 2),
      sinc_damping.T.reshape(K, NCHUNK, A).transpose(1, 0, 2),
      batch_row, batch_col, W_pre1.T, W_pre2.T, gamma2, beta2, kf3,
      e1t, e1b, W_upd1.T, W_upd2.T)

    return out
